# Initial kernel scaffold; baseline (speedup 1.0000x reference)
import jax
import jax.numpy as jnp
from jax.experimental import pallas as pl


def kernel(x_num, x_cat, num_weights, num_biases, cat_tables, cat_biases, cls_token):
    raise NotImplementedError("write your pallas kernel here")


# speedup vs baseline: 15.8409x; 15.8409x over previous
import functools

import jax
import jax.numpy as jnp
from jax import lax
from jax.experimental import pallas as pl
from jax.experimental.pallas import tpu as pltpu
from jax.experimental.pallas import tpu_sc as plsc

_NC = 2
_NS = 16
_L = 16
_NW = _NC * _NS

_B = 16384
_NNUM = 13
_NCAT = 26
_CATDIM = 1000
_D = 128
_NSLOT = 1 + _NNUM + _NCAT

_BPW = _B // _NW
_CB = 16
_NCHUNK = _BPW // _CB

_QS = _D // _L

_mesh = plsc.VectorSubcoreMesh(core_axis_name="c", subcore_axis_name="s")


@functools.partial(
    pl.kernel,
    mesh=_mesh,
    compiler_params=pltpu.CompilerParams(needs_layout_passes=False),
    out_type=jax.ShapeDtypeStruct((_B, _NSLOT, _D), jnp.float32),
    scratch_types=[
        pltpu.VMEM((_CB, _NCAT), jnp.int32),
        pltpu.VMEM((_CB, _NSLOT, _D), jnp.float32),
        pltpu.VMEM((_CB * _NNUM,), jnp.float32),
        pltpu.VMEM((_NNUM, _D), jnp.float32),
        pltpu.VMEM((_NNUM, _D), jnp.float32),
        pltpu.VMEM((1, _D), jnp.float32),
        pltpu.SemaphoreType.DMA,
        pltpu.SemaphoreType.DMA,
        pltpu.SemaphoreType.DMA,
    ],
)
def _tokenize_sc(xnum_hbm, idx_hbm, w_hbm, bn_hbm, tab_hbm, cls_hbm, out_hbm,
                 idx_v, stage_buf, xn_v, w_v, bn_v, cls_v,
                 gsem, osem, ssem):
    wid = lax.axis_index("s") * _NC + lax.axis_index("c")
    base0 = wid * _BPW

    pltpu.async_copy(w_hbm, w_v, ssem).wait()
    pltpu.async_copy(bn_hbm, bn_v, ssem).wait()
    pltpu.async_copy(cls_hbm, cls_v, ssem).wait()

    for q in range(_QS):
        cv = cls_v[0, pl.ds(q * _L, _L)]

        def fill_b(b, _, cv=cv, q=q):
            stage_buf[b, 0, pl.ds(q * _L, _L)] = cv
            return 0

        lax.fori_loop(0, _CB, fill_b, 0)

    def chunk_body(c, _):
        base = base0 + c * _CB
        pltpu.async_copy(idx_hbm.at[pl.ds(base, _CB)], idx_v, ssem).wait()
        pltpu.async_copy(xnum_hbm.at[pl.ds(base * _NNUM, _CB * _NNUM)],
                         xn_v, ssem).wait()

        gathers = []
        for b in range(_CB):
            gathers.append(
                pltpu.async_copy(tab_hbm.at[idx_v.at[b]],
                                 stage_buf.at[b, pl.ds(1 + _NNUM, _NCAT)],
                                 gsem))

        def num_body(b, _):
            for j in range(_NNUM):
                fi = jnp.full((_L,), b * _NNUM + j, jnp.int32)
                xs = plsc.load_gather(xn_v, [fi])
                for q in range(_QS):
                    w = w_v[j, pl.ds(q * _L, _L)]
                    bb = bn_v[j, pl.ds(q * _L, _L)]
                    stage_buf[b, 1 + j, pl.ds(q * _L, _L)] = xs * w + bb
            return 0

        lax.fori_loop(0, _CB, num_body, 0)

        for g in gathers:
            g.wait()

        pltpu.async_copy(stage_buf, out_hbm.at[pl.ds(base, _CB)], osem).wait()
        return 0

    lax.fori_loop(0, _NCHUNK, chunk_body, 0)


def kernel(x_num, x_cat, num_weights, num_biases, cat_tables, cat_biases,
           cls_token):
    offs = (jnp.arange(_NCAT, dtype=jnp.int32) * _CATDIM)[None, :]
    idx = x_cat.astype(jnp.int32) + offs
    tab = (cat_tables + cat_biases[:, None, :]).reshape(_NCAT * _CATDIM, _D)
    cls = cls_token.reshape(1, _D)
    return _tokenize_sc(x_num.reshape(-1), idx, num_weights, num_biases, tab,
                        cls)


# speedup vs baseline: 21.8022x; 1.3763x over previous
import functools

import jax
import jax.numpy as jnp
from jax import lax
from jax.experimental import pallas as pl
from jax.experimental.pallas import tpu as pltpu
from jax.experimental.pallas import tpu_sc as plsc

_NC = 2
_NS = 16
_L = 16
_NW = _NC * _NS

_B = 16384
_NNUM = 13
_NCAT = 26
_CATDIM = 1000
_D = 128
_NSLOT = 1 + _NNUM + _NCAT

_BPW = _B // _NW
_CB = 8
_NCHUNK = _BPW // _CB

_QS = _D // _L

_mesh = plsc.VectorSubcoreMesh(core_axis_name="c", subcore_axis_name="s")


@functools.partial(
    pl.kernel,
    mesh=_mesh,
    compiler_params=pltpu.CompilerParams(needs_layout_passes=False),
    out_type=jax.ShapeDtypeStruct((_B, _NSLOT, _D), jnp.float32),
    scratch_types=[
        pltpu.VMEM((2, _CB, _NCAT), jnp.int32),
        pltpu.VMEM((2, _CB, _NSLOT, _D), jnp.float32),
        pltpu.VMEM((2, _CB * _NNUM), jnp.float32),
        pltpu.VMEM((_NNUM, _D), jnp.float32),
        pltpu.VMEM((_NNUM, _D), jnp.float32),
        pltpu.VMEM((1, _D), jnp.float32),
        pltpu.SemaphoreType.DMA,
        pltpu.SemaphoreType.DMA,
        pltpu.SemaphoreType.DMA,
    ],
)
def _tokenize_sc(xnum_hbm, idx_hbm, w_hbm, bn_hbm, tab_hbm, cls_hbm, out_hbm,
                 idx_v, stage_buf, xn_v, w_v, bn_v, cls_v,
                 gsem, osem, ssem):
    wid = lax.axis_index("s") * _NC + lax.axis_index("c")
    base0 = wid * _BPW

    def issue_stage(c, par):
        base = base0 + c * _CB
        pltpu.async_copy(idx_hbm.at[pl.ds(base, _CB)], idx_v.at[par], ssem)
        pltpu.async_copy(xnum_hbm.at[pl.ds(base * _NNUM, _CB * _NNUM)],
                         xn_v.at[par], ssem)

    def wait_stage(par):
        pltpu.make_async_copy(idx_hbm.at[pl.ds(0, _CB)], idx_v.at[par],
                              ssem).wait()
        pltpu.make_async_copy(xnum_hbm.at[pl.ds(0, _CB * _NNUM)],
                              xn_v.at[par], ssem).wait()

    def wait_out(par):
        pltpu.make_async_copy(stage_buf.at[par], out_hbm.at[pl.ds(0, _CB)],
                              osem).wait()

    pltpu.async_copy(w_hbm, w_v, ssem).wait()
    pltpu.async_copy(bn_hbm, bn_v, ssem).wait()
    pltpu.async_copy(cls_hbm, cls_v, ssem).wait()

    for q in range(_QS):
        cv = cls_v[0, pl.ds(q * _L, _L)]
        for p in range(2):

            def fill_b(b, _, cv=cv, q=q, p=p):
                stage_buf[p, b, 0, pl.ds(q * _L, _L)] = cv
                return 0

            lax.fori_loop(0, _CB, fill_b, 0)

    issue_stage(0, 0)

    def chunk_body(c, _):
        par = lax.rem(c, 2)
        base = base0 + c * _CB

        wait_stage(par)

        @pl.when(c >= 2)
        def _():
            wait_out(par)

        gathers = []
        for b in range(_CB):
            gathers.append(
                pltpu.async_copy(tab_hbm.at[idx_v.at[par, b]],
                                 stage_buf.at[par, b, pl.ds(1 + _NNUM, _NCAT)],
                                 gsem))

        @pl.when(c + 1 < _NCHUNK)
        def _():
            issue_stage(c + 1, 1 - par)

        def num_body(b, _):
            for j in range(_NNUM):
                fi = jnp.full((_L,), b * _NNUM + j, jnp.int32)
                xs = plsc.load_gather(xn_v.at[par], [fi])
                for q in range(_QS):
                    w = w_v[j, pl.ds(q * _L, _L)]
                    bb = bn_v[j, pl.ds(q * _L, _L)]
                    stage_buf[par, b, 1 + j, pl.ds(q * _L, _L)] = xs * w + bb
            return 0

        lax.fori_loop(0, _CB, num_body, 0)

        for g in gathers:
            g.wait()

        pltpu.async_copy(stage_buf.at[par], out_hbm.at[pl.ds(base, _CB)], osem)
        return 0

    lax.fori_loop(0, _NCHUNK, chunk_body, 0)

    wait_out(0)
    wait_out(1)


def kernel(x_num, x_cat, num_weights, num_biases, cat_tables, cat_biases,
           cls_token):
    offs = (jnp.arange(_NCAT, dtype=jnp.int32) * _CATDIM)[None, :]
    idx = x_cat.astype(jnp.int32) + offs
    tab = (cat_tables + cat_biases[:, None, :]).reshape(_NCAT * _CATDIM, _D)
    cls = cls_token.reshape(1, _D)
    return _tokenize_sc(x_num.reshape(-1), idx, num_weights, num_biases, tab,
                        cls)
